# Initial kernel scaffold; baseline (speedup 1.0000x reference)
import jax
import jax.numpy as jnp
from jax.experimental import pallas as pl


def kernel(positions, hash_table):
    raise NotImplementedError("write your pallas kernel here")


# speedup vs baseline: 1.8931x; 1.8931x over previous
import math

import jax
import jax.numpy as jnp
from jax import lax
from jax.experimental import pallas as pl
from jax.experimental.pallas import tpu as pltpu
from jax.experimental.pallas import tpu_sc as plsc

LEVELS = 8
BASE_RES = 16.0
MAX_RES = 2048.0
FEAT = 2
MAX_PARAMS = 2 ** 19
LOG_B = math.log(MAX_RES / BASE_RES) / (LEVELS - 1)

_OFFS = []
_SIZES = []
_RES = []
_SCALES = []
_off = 0
_FIRST_HASHED = LEVELS
for _i in range(LEVELS):
    _scale = BASE_RES * math.exp(_i * LOG_B) - 1.0
    _res = int(math.ceil(_scale)) + 1
    _full = _res ** 3
    _full_aligned = ((_full + 7) // 8) * 8
    _sz = min(MAX_PARAMS, _full_aligned)
    _OFFS.append(_off)
    _SIZES.append(_sz)
    _RES.append(_res)
    _SCALES.append(_scale)
    if _full > _sz and _FIRST_HASHED == LEVELS:
        _FIRST_HASHED = _i
    _off += _sz
TOTAL_ROWS = _off
for _s in _SIZES:
    assert _s & (_s - 1) == 0, "level sizes must be powers of two"

P1 = 2654435761 - 2 ** 32
P2 = 805459861

B = 524288
NC, NS = 2, 16
NW = NC * NS
PB = B // NW
C = 128
NV = C // 16
NCHUNK = PB // C
N_LOCAL_LVL = 2
LOCAL_FLOATS = _OFFS[N_LOCAL_LVL] * FEAT
N_LC = LEVELS * 8
LOCAL_LC = N_LOCAL_LVL * 8
HBM_LC = N_LC - LOCAL_LC


def _sc_body(pos_hbm, tab_hbm, out_hbm,
             lvl01, pos_b, idx_b, w_b, rows_b, acc_b, out_b, sem_g):
    wid = lax.axis_index("s") * NC + lax.axis_index("c")
    tile_base = wid * PB

    pltpu.sync_copy(tab_hbm.at[pl.ds(0, LOCAL_FLOATS)], lvl01)

    iota = lax.iota(jnp.int32, 16)
    pair_sel = iota >> 1
    f_idx = iota & 1
    zeros16 = jnp.zeros((16,), jnp.float32)

    def chunk(ci, _):
        base = tile_base + ci * C
        pltpu.sync_copy(pos_hbm.at[:, pl.ds(base, C)], pos_b)

        def zero_body(i, _):
            acc_b[i >> 4, (i >> 3) & 1, pl.ds((i & 7) * 16, 16)] = zeros16
            return _
        lax.fori_loop(0, LEVELS * 2 * NV, zero_body, None)

        handles = []
        for l in range(LEVELS):
            scale = _SCALES[l]
            mask = _SIZES[l] - 1
            off2 = _OFFS[l] * 2
            res = _RES[l]

            def lvl_body(v, _, scale=scale, mask=mask, off2=off2, res=res, l=l):
                px = pos_b[0, pl.ds(v * 16, 16)] * scale + 0.5
                py = pos_b[1, pl.ds(v * 16, 16)] * scale + 0.5
                pz = pos_b[2, pl.ds(v * 16, 16)] * scale + 0.5
                gx = px.astype(jnp.int32)
                gy = py.astype(jnp.int32)
                gz = pz.astype(jnp.int32)
                fx = px - gx.astype(jnp.float32)
                fy = py - gy.astype(jnp.float32)
                fz = pz - gz.astype(jnp.float32)
                if l < _FIRST_HASHED:
                    ax = (gx, gx + 1)
                    ay = (gy * res, gy * res + res)
                    az = (gz * (res * res), gz * (res * res) + res * res)
                    comb = lambda a, b: a + b
                else:
                    ax = (gx, gx + 1)
                    ay = (gy * P1, gy * P1 + P1)
                    az = (gz * P2, gz * P2 + P2)
                    comb = lax.bitwise_xor
                wx = (1.0 - fx, fx)
                wy = (1.0 - fy, fy)
                wz = (1.0 - fz, fz)
                wxy = [wx[0] * wy[0], wx[1] * wy[0], wx[0] * wy[1], wx[1] * wy[1]]
                for c in range(8):
                    cx, cy, cz = c & 1, (c >> 1) & 1, c >> 2
                    h = comb(comb(ax[cx], ay[cy]), az[cz])
                    i0 = ((h & mask) << 1) + off2
                    lc = l * 8 + c
                    idx_b[2 * lc, pl.ds(v * 16, 16)] = i0
                    idx_b[2 * lc + 1, pl.ds(v * 16, 16)] = i0 + 1
                    w_b[lc, pl.ds(v * 16, 16)] = wxy[cy * 2 + cx] * wz[cz]
                return _

            lax.fori_loop(0, NV, lvl_body, None)

            if l >= N_LOCAL_LVL:
                for c in range(8):
                    lc = l * 8 + c
                    for f in range(2):
                        handles.append(pltpu.async_copy(
                            tab_hbm.at[idx_b.at[2 * lc + f]],
                            rows_b.at[2 * (lc - LOCAL_LC) + f], sem_g))

        def local_body(i, _):
            lc = i >> 3
            v = i & 7
            sl = pl.ds(v * 16, 16)
            f0 = plsc.load_gather(lvl01, [idx_b[2 * lc, sl]])
            f1 = plsc.load_gather(lvl01, [idx_b[2 * lc + 1, sl]])
            wv = w_b[lc, sl]
            plsc.addupdate(acc_b.at[lc >> 3, 0, sl], wv * f0)
            plsc.addupdate(acc_b.at[lc >> 3, 1, sl], wv * f1)
            return _
        lax.fori_loop(0, LOCAL_LC * NV, local_body, None)

        for h in handles:
            h.wait()

        def hbm_body(i, _):
            lc16 = i >> 3
            v = i & 7
            lc = lc16 + LOCAL_LC
            sl = pl.ds(v * 16, 16)
            f0 = rows_b[2 * lc16, sl]
            f1 = rows_b[2 * lc16 + 1, sl]
            wv = w_b[lc, sl]
            plsc.addupdate(acc_b.at[lc >> 3, 0, sl], wv * f0)
            plsc.addupdate(acc_b.at[lc >> 3, 1, sl], wv * f1)
            return _
        lax.fori_loop(0, HBM_LC * NV, hbm_body, None)

        def asm_body(r, _):
            row = plsc.load_gather(acc_b, [pair_sel, f_idx, jnp.full((16,), r, jnp.int32)])
            out_b[r] = row
            return _
        lax.fori_loop(0, C, asm_body, None)

        pltpu.sync_copy(out_b, out_hbm.at[pl.ds(base, C)])
        return _

    lax.fori_loop(0, NCHUNK, chunk, None)


@jax.jit
def _encode_sc(positions_t, table_flat):
    mesh = plsc.VectorSubcoreMesh(core_axis_name="c", subcore_axis_name="s")
    return pl.kernel(
        _sc_body,
        out_type=jax.ShapeDtypeStruct((B, LEVELS * FEAT), jnp.float32),
        mesh=mesh,
        compiler_params=pltpu.CompilerParams(
            needs_layout_passes=False, use_tc_tiling_on_sc=False),
        scratch_types=[
            pltpu.VMEM((LOCAL_FLOATS,), jnp.float32),
            pltpu.VMEM((3, C), jnp.float32),
            pltpu.VMEM((2 * N_LC, C), jnp.int32),
            pltpu.VMEM((N_LC, C), jnp.float32),
            pltpu.VMEM((2 * HBM_LC, C), jnp.float32),
            pltpu.VMEM((LEVELS, 2, C), jnp.float32),
            pltpu.VMEM((C, LEVELS * FEAT), jnp.float32),
            pltpu.SemaphoreType.DMA,
        ],
    )(positions_t, table_flat)


def kernel(positions, hash_table):
    return _encode_sc(positions.T, hash_table)
